# baseline (speedup 1.0000x reference)
import functools

import jax
import jax.numpy as jnp
from jax import lax
from jax.experimental import pallas as pl
from jax.experimental.pallas import tpu as pltpu
from jax.experimental.pallas import tpu_sc as plsc

NC, NS = 2, 16
NW = NC * NS
NI = 4096
NJ = 200
D = 64
VOCAB2 = 500000
IB = NI // NW
JB = 8
NR = NJ // JB

_mesh = plsc.VectorSubcoreMesh(core_axis_name="c", subcore_axis_name="s")


@functools.partial(
    pl.kernel,
    out_type=jax.ShapeDtypeStruct((NJ, D, NI), jnp.float32),
    mesh=_mesh,
    compiler_params=pltpu.CompilerParams(
        use_tc_tiling_on_sc=True, needs_layout_passes=False),
    scratch_types=[
        pltpu.VMEM((JB, IB), jnp.int32),
        pltpu.VMEM((IB,), jnp.int32),
        pltpu.VMEM((IB,), jnp.int32),
        pltpu.VMEM((IB,), jnp.int32),
        pltpu.VMEM((IB,), jnp.int32),
        pltpu.VMEM((IB, 128), jnp.float32),
        pltpu.VMEM((IB, 128), jnp.float32),
        pltpu.VMEM((D, IB), jnp.float32),
        pltpu.VMEM((D, IB), jnp.float32),
        pltpu.SemaphoreType.DMA,
        pltpu.SemaphoreType.DMA,
    ],
)
def _emb_lookup(xt_hbm, tbl2_hbm, out_hbm,
                xtile, idx0, idx1, hb0, hb1, pr0, pr1, ot0, ot1, sem0, sem1):
    w = lax.axis_index("s") * NC + lax.axis_index("c")
    i0 = w * IB
    idx = (idx0, idx1)
    hb = (hb0, hb1)
    pairs = (pr0, pr1)
    outt = (ot0, ot1)
    sem = (sem0, sem1)

    def fire(jj, b):
        for t in range(IB // 16):
            v = xtile[jj, pl.ds(16 * t, 16)]
            idx[b][pl.ds(16 * t, 16)] = lax.shift_right_logical(v, 1)
            hb[b][pl.ds(16 * t, 16)] = lax.shift_left(jnp.bitwise_and(v, 1), 6)
        pltpu.async_copy(tbl2_hbm.at[idx[b]], pairs[b], sem[b])

    def drain_transpose_write(r, jj, b):
        pltpu.make_async_copy(tbl2_hbm.at[pl.ds(0, IB)], pairs[b], sem[b]).wait()

        def body(kk, carry):
            for t in range(IB // 16):
                rows = lax.iota(jnp.int32, 16) + (16 * t)
                cols = hb[b][pl.ds(16 * t, 16)] + kk
                outt[b][kk, pl.ds(16 * t, 16)] = plsc.load_gather(
                    pairs[b], [rows, cols])
            return carry

        lax.fori_loop(0, D, body, 0, unroll=False)
        pltpu.sync_copy(outt[b], out_hbm.at[r * JB + jj, :, pl.ds(i0, IB)])

    def block(r, carry):
        pltpu.sync_copy(xt_hbm.at[pl.ds(r * JB, JB), pl.ds(i0, IB)], xtile)
        fire(0, 0)
        for jj in range(JB):
            b = jj % 2
            if jj + 1 < JB:
                fire(jj + 1, 1 - b)
            drain_transpose_write(r, jj, b)
        return carry

    lax.fori_loop(0, NR, block, 0)


def kernel(x, table):
    out_t = _emb_lookup(x.T, jnp.reshape(table, (VOCAB2, 128)))
    return jnp.transpose(out_t, (2, 0, 1))
